# Initial kernel scaffold; baseline (speedup 1.0000x reference)
import functools

import jax
import jax.numpy as jnp
from jax import lax
from jax.experimental import pallas as pl
from jax.experimental.pallas import tpu as pltpu
from jax.experimental.pallas import tpu_sc as plsc

H = 128
D = 128
N = 10000
E = 320000
G = 128
C = 2
L = 3

NC = 2
NS = 16
NW = NC * NS
PERW = E // NW
K = 80
NCHUNK = PERW // K
RPT = N // NS

F32 = jnp.float32
I32 = jnp.int32


def _mesh():
    return plsc.VectorSubcoreMesh(
        core_axis_name="c", subcore_axis_name="s", num_cores=NC, num_subcores=NS
    )


def _sc_segsum(h, src, dst, zrows):
    @functools.partial(
        pl.kernel,
        out_type=jax.ShapeDtypeStruct((NC, N, H), F32),
        mesh=_mesh(),
        scratch_types=[
            pltpu.VMEM((K,), I32),
            pltpu.VMEM((K,), I32),
            pltpu.VMEM((K, H), F32),
            pltpu.VMEM_SHARED((N, H), F32),
            pltpu.SemaphoreType.DMA,
        ],
    )
    def k(h_hbm, src_hbm, dst_hbm, z_hbm, out_hbm, sidx, didx, rows, acc, sem):
        c = lax.axis_index("c")
        s = lax.axis_index("s")
        pltpu.sync_copy(z_hbm, acc.at[pl.ds(s * RPT, RPT)])
        plsc.subcore_barrier()
        base = (c * NS + s) * PERW

        def body(j, carry):
            off = base + j * K
            pltpu.sync_copy(src_hbm.at[pl.ds(off, K)], sidx)
            pltpu.sync_copy(dst_hbm.at[pl.ds(off, K)], didx)
            pltpu.async_copy(h_hbm.at[sidx], rows, sem).wait()
            pltpu.sync_copy(rows, acc.at[didx], add=True)
            return carry

        lax.fori_loop(0, NCHUNK, body, 0)
        plsc.subcore_barrier()
        pltpu.sync_copy(
            acc.at[pl.ds(s * RPT, RPT)], out_hbm.at[c, pl.ds(s * RPT, RPT)]
        )

    return k(h, src, dst, zrows)


def _sc_edge_weights(eaa, eab, src, dst, zdeg):
    @functools.partial(
        pl.kernel,
        out_type=(
            jax.ShapeDtypeStruct((E,), F32),
            jax.ShapeDtypeStruct((E,), F32),
            jax.ShapeDtypeStruct((NC, N, 16), F32),
        ),
        mesh=_mesh(),
        scratch_types=[
            pltpu.VMEM((N, 2), F32),
            pltpu.VMEM((N, 2), F32),
            pltpu.VMEM((K,), I32),
            pltpu.VMEM((K,), I32),
            pltpu.VMEM((K,), F32),
            pltpu.VMEM((K,), F32),
            pltpu.VMEM((K, 16), F32),
            pltpu.VMEM_SHARED((N, 16), F32),
        ],
    )
    def k(eaa_hbm, eab_hbm, src_hbm, dst_hbm, z_hbm, wc_hbm, wo_hbm, deg_hbm,
          eaa_v, eab_v, sidx, didx, wcb, wob, degrows, dacc):
        c = lax.axis_index("c")
        s = lax.axis_index("s")
        pltpu.sync_copy(z_hbm, dacc.at[pl.ds(s * RPT, RPT)])
        pltpu.sync_copy(eaa_hbm, eaa_v)
        pltpu.sync_copy(eab_hbm, eab_v)
        zero16 = jnp.zeros((16,), F32)
        for r in range(K):
            degrows[r, :] = zero16
        plsc.subcore_barrier()
        base = (c * NS + s) * PERW
        iota16 = lax.iota(I32, 16)
        zi = jnp.zeros((16,), I32)
        oi = jnp.ones((16,), I32)
        ones_f = jnp.ones((16,), F32)
        zeros_f = jnp.zeros((16,), F32)

        def body(j, carry):
            off = base + j * K
            pltpu.sync_copy(src_hbm.at[pl.ds(off, K)], sidx)
            pltpu.sync_copy(dst_hbm.at[pl.ds(off, K)], didx)
            for g in range(K // 16):
                r = sidx[pl.ds(g * 16, 16)]
                cc = didx[pl.ds(g * 16, 16)]
                a0 = plsc.load_gather(eaa_v, [r, zi])
                a1 = plsc.load_gather(eaa_v, [r, oi])
                b0 = plsc.load_gather(eab_v, [cc, zi])
                b1 = plsc.load_gather(eab_v, [cc, oi])
                l0 = a0 + b0
                l1 = a1 + b1
                m = jnp.maximum(l0, l1)
                e0 = jnp.exp(l0 - m)
                e1 = jnp.exp(l1 - m)
                inv = 1.0 / (e0 + e1)
                keep = jnp.where(r == cc, zeros_f, ones_f)
                wcv = e0 * inv * keep
                wov = e1 * inv * keep
                wcb[pl.ds(g * 16, 16)] = wcv
                wob[pl.ds(g * 16, 16)] = wov
                rowids = g * 16 + iota16
                plsc.store_scatter(degrows, [rowids, zi], wcv)
                plsc.store_scatter(degrows, [rowids, oi], wov)
            pltpu.sync_copy(wcb, wc_hbm.at[pl.ds(off, K)])
            pltpu.sync_copy(wob, wo_hbm.at[pl.ds(off, K)])
            pltpu.sync_copy(degrows, dacc.at[sidx], add=True)
            return carry

        lax.fori_loop(0, NCHUNK, body, 0)
        plsc.subcore_barrier()
        pltpu.sync_copy(
            dacc.at[pl.ds(s * RPT, RPT)], deg_hbm.at[c, pl.ds(s * RPT, RPT)]
        )

    return k(eaa, eab, src, dst, zdeg)


def _sc_gcn_agg(y, w, src, dst, zrows):
    @functools.partial(
        pl.kernel,
        out_type=jax.ShapeDtypeStruct((NC, N, H), F32),
        mesh=_mesh(),
        scratch_types=[
            pltpu.VMEM((K,), I32),
            pltpu.VMEM((K,), I32),
            pltpu.VMEM((K,), F32),
            pltpu.VMEM((K, H), F32),
            pltpu.VMEM_SHARED((N, H), F32),
            pltpu.SemaphoreType.DMA,
        ],
    )
    def k(y_hbm, w_hbm, src_hbm, dst_hbm, z_hbm, out_hbm,
          sidx, didx, wbuf, rows, acc, sem):
        c = lax.axis_index("c")
        s = lax.axis_index("s")
        pltpu.sync_copy(z_hbm, acc.at[pl.ds(s * RPT, RPT)])
        plsc.subcore_barrier()
        base = (c * NS + s) * PERW

        def body(j, carry):
            off = base + j * K
            pltpu.sync_copy(src_hbm.at[pl.ds(off, K)], sidx)
            pltpu.sync_copy(dst_hbm.at[pl.ds(off, K)], didx)
            pltpu.sync_copy(w_hbm.at[pl.ds(off, K)], wbuf)
            pltpu.async_copy(y_hbm.at[sidx], rows, sem).wait()

            def scale(r, inner):
                sv = wbuf[r]
                for q in range(H // 16):
                    sl = pl.ds(q * 16, 16)
                    rows[r, sl] = rows[r, sl] * sv
                return inner

            lax.fori_loop(0, K, scale, 0)
            pltpu.sync_copy(rows, acc.at[didx], add=True)
            return carry

        lax.fori_loop(0, NCHUNK, body, 0)
        plsc.subcore_barrier()
        pltpu.sync_copy(
            acc.at[pl.ds(s * RPT, RPT)], out_hbm.at[c, pl.ds(s * RPT, RPT)]
        )

    return k(y, w, src, dst, zrows)


def _bn_in(x):
    m = jnp.mean(x, axis=0, keepdims=True)
    xc = x - m
    v = jnp.mean(xc * xc, axis=0, keepdims=True)
    return xc * lax.rsqrt(v + 1e-5) + 1e-4


def _tc_feat(x, wf):
    def body(x_ref, w_ref, o_ref):
        h = _bn_in(x_ref[...])
        o_ref[...] = jnp.maximum(
            jnp.dot(h, w_ref[...], preferred_element_type=F32), 0.0
        )

    return pl.pallas_call(
        body, out_shape=jax.ShapeDtypeStruct((N, H), F32)
    )(x, wf)


def _tc_gin(hs, agg, w1, b1, w2, b2):
    def body(hs_ref, a_ref, w1_ref, b1_ref, w2_ref, b2_ref, h_ref, h2_ref):
        t = hs_ref[...] + a_ref[0] + a_ref[1]
        u = jnp.dot(t, w1_ref[...], preferred_element_type=F32) + b1_ref[...]
        u = jnp.maximum(_bn_in(u), 0.0)
        v = jnp.dot(u, w2_ref[...], preferred_element_type=F32) + b2_ref[...]
        h = jnp.maximum(v, 0.0)
        h_ref[...] = h
        h2_ref[...] = 2.0 * h

    return pl.pallas_call(
        body,
        out_shape=(
            jax.ShapeDtypeStruct((N, H), F32),
            jax.ShapeDtypeStruct((N, H), F32),
        ),
    )(hs, agg, w1, b1, w2, b2)


def _tc_att(h, wea, bea, wna, bna, wctx, wobj):
    def body(h_ref, wea_ref, bea_ref, wna_ref, bna_ref, wc_ref, wo_ref,
             xcw_ref, xow_ref, eaa_ref, eab_ref):
        h = h_ref[...]
        nl = jnp.dot(h, wna_ref[...], preferred_element_type=F32) + bna_ref[...]
        mx = jnp.max(nl, axis=-1, keepdims=True)
        ex = jnp.exp(nl - mx)
        na = ex / jnp.sum(ex, axis=-1, keepdims=True)
        xc = na[:, 0:1] * h
        xo = na[:, 1:2] * h
        xcw_ref[...] = jnp.dot(_bn_in(xc), wc_ref[...], preferred_element_type=F32)
        xow_ref[...] = jnp.dot(_bn_in(xo), wo_ref[...], preferred_element_type=F32)
        wea = wea_ref[...]
        eaa_ref[...] = (
            jnp.dot(h, wea[:H, :], preferred_element_type=F32) + bea_ref[...]
        )
        eab_ref[...] = jnp.dot(h, wea[H:, :], preferred_element_type=F32)

    return pl.pallas_call(
        body,
        out_shape=(
            jax.ShapeDtypeStruct((N, H), F32),
            jax.ShapeDtypeStruct((N, H), F32),
            jax.ShapeDtypeStruct((N, 2), F32),
            jax.ShapeDtypeStruct((N, 2), F32),
        ),
    )(h, wea, bea, wna, bna, wctx, wobj)


def _tc_deg(dacc, xcw, xow):
    def body(d_ref, xc_ref, xo_ref, yc_ref, yo_ref, dc_ref, do_ref):
        degc = 1.0 + d_ref[0, :, 0:1] + d_ref[1, :, 0:1]
        dego = 1.0 + d_ref[0, :, 1:2] + d_ref[1, :, 1:2]
        disc = lax.rsqrt(jnp.maximum(degc, 1e-12))
        diso = lax.rsqrt(jnp.maximum(dego, 1e-12))
        yc_ref[...] = disc * xc_ref[...]
        yo_ref[...] = diso * xo_ref[...]
        dc_ref[...] = disc
        do_ref[...] = diso

    return pl.pallas_call(
        body,
        out_shape=(
            jax.ShapeDtypeStruct((N, H), F32),
            jax.ShapeDtypeStruct((N, H), F32),
            jax.ShapeDtypeStruct((N, 1), F32),
            jax.ShapeDtypeStruct((N, 1), F32),
        ),
    )(dacc, xcw, xow)


def _tc_pool(accc, yc, dc, bctx, acco, yo, do_, bobj, batch2d):
    def body(ac_ref, yc_ref, dc_ref, bc_ref, ao_ref, yo_ref, do_ref, bo_ref,
             b_ref, pc_ref, po_ref):
        xc = jnp.maximum(
            dc_ref[...] * (ac_ref[0] + ac_ref[1] + yc_ref[...]) + bc_ref[...], 0.0
        )
        xo = jnp.maximum(
            do_ref[...] * (ao_ref[0] + ao_ref[1] + yo_ref[...]) + bo_ref[...], 0.0
        )
        oh = (b_ref[...] == lax.broadcasted_iota(I32, (N, G), 1)).astype(F32)
        pc_ref[...] = lax.dot_general(
            oh, xc, (((0,), (0,)), ((), ())), preferred_element_type=F32
        )
        po_ref[...] = lax.dot_general(
            oh, xo, (((0,), (0,)), ((), ())), preferred_element_type=F32
        )

    return pl.pallas_call(
        body,
        out_shape=(
            jax.ShapeDtypeStruct((G, H), F32),
            jax.ShapeDtypeStruct((G, H), F32),
        ),
    )(accc, yc, dc, bctx, acco, yo, do_, bobj, batch2d)


def _tc_read(pc, po, permoh, ws):
    def body(pc_ref, po_ref, perm_ref,
             w1c, b1c, w2c, b2c, w1o, b1o, w2o, b2o, w1x, b1x, w2x, b2x,
             oc_ref, oo_ref, ox_ref):
        def readout(z, w1, b1, w2, b2):
            z = _bn_in(z)
            z = jnp.maximum(
                jnp.dot(z, w1[...], preferred_element_type=F32) + b1[...], 0.0
            )
            z = _bn_in(z)
            z = jnp.dot(z, w2[...], preferred_element_type=F32) + b2[...]
            mx = jnp.max(z, axis=-1, keepdims=True)
            ez = jnp.exp(z - mx)
            return z - mx - jnp.log(jnp.sum(ez, axis=-1, keepdims=True))

        pcv = pc_ref[...]
        pov = po_ref[...]
        oc_ref[...] = readout(pcv, w1c, b1c, w2c, b2c)
        oo_ref[...] = readout(pov, w1o, b1o, w2o, b2o)
        xco = jnp.dot(perm_ref[...], pcv, preferred_element_type=F32) + pov
        ox_ref[...] = readout(xco, w1x, b1x, w2x, b2x)

    return pl.pallas_call(
        body,
        out_shape=(
            jax.ShapeDtypeStruct((G, C), F32),
            jax.ShapeDtypeStruct((G, C), F32),
            jax.ShapeDtypeStruct((G, C), F32),
        ),
    )(pc, po, permoh, *ws)


def kernel(x, edge_index, batch, params):
    row = edge_index[0].astype(I32)
    col = edge_index[1].astype(I32)
    batch2d = batch.astype(I32).reshape(N, 1)
    p = params

    def b2(v):
        return v.reshape(1, -1)

    zrows = jnp.zeros((RPT, H), F32)
    zdeg = jnp.zeros((RPT, 16), F32)

    h = _tc_feat(x, p["W_feat"])
    hs = h
    for i in range(L):
        agg = _sc_segsum(hs, row, col, zrows)
        h, hs = _tc_gin(
            hs, agg,
            p["gin%d_W1" % i], b2(p["gin%d_b1" % i]),
            p["gin%d_W2" % i], b2(p["gin%d_b2" % i]),
        )

    xcw, xow, eaa, eab = _tc_att(
        h, p["W_ea"], b2(p["b_ea"]), p["W_na"], b2(p["b_na"]),
        p["W_ctx"], p["W_obj"],
    )
    wc, wo, dacc = _sc_edge_weights(eaa, eab, row, col, zdeg)
    yc, yo, dc, do_ = _tc_deg(dacc, xcw, xow)
    accc = _sc_gcn_agg(yc, wc, row, col, zrows)
    acco = _sc_gcn_agg(yo, wo, row, col, zrows)
    pc, po = _tc_pool(
        accc, yc, dc, b2(p["b_ctx"]), acco, yo, do_, b2(p["b_obj"]), batch2d
    )

    perm = jax.random.permutation(jax.random.key(1), G)
    permoh = jax.nn.one_hot(perm, G, dtype=F32)
    ws = (
        p["fc1_c_W"], b2(p["fc1_c_b"]), p["fc2_c_W"], b2(p["fc2_c_b"]),
        p["fc1_o_W"], b2(p["fc1_o_b"]), p["fc2_o_W"], b2(p["fc2_o_b"]),
        p["fc1_co_W"], b2(p["fc1_co_b"]), p["fc2_co_W"], b2(p["fc2_co_b"]),
    )
    return _tc_read(pc, po, permoh, ws)


# speedup vs baseline: 6.6331x; 6.6331x over previous
import functools

import jax
import jax.numpy as jnp
from jax import lax
from jax.experimental import pallas as pl
from jax.experimental.pallas import tpu as pltpu
from jax.experimental.pallas import tpu_sc as plsc

H = 128
D = 128
N = 10000
E = 320000
G = 128
C = 2
L = 3

NC = 2
NS = 16
NW = NC * NS
PERW = E // NW
K = 80
NCHUNK = PERW // K
NP = 10240
RPT = NP // NS

F32 = jnp.float32
BF16 = jnp.bfloat16
I32 = jnp.int32


def _dot3(a, b):
    return jnp.dot(a, b, preferred_element_type=F32)


def _mesh():
    return plsc.VectorSubcoreMesh(
        core_axis_name="c", subcore_axis_name="s", num_cores=NC, num_subcores=NS
    )


def _sc_segsum(h, src, dst, zrows):
    @functools.partial(
        pl.kernel,
        out_type=jax.ShapeDtypeStruct((NC, NP, H), F32),
        mesh=_mesh(),
        scratch_types=[
            pltpu.VMEM((K,), I32),
            pltpu.VMEM((K,), I32),
            pltpu.VMEM((K, H), F32),
            pltpu.VMEM_SHARED((NP, H), F32),
            pltpu.SemaphoreType.DMA,
        ],
    )
    def k(h_hbm, src_hbm, dst_hbm, z_hbm, out_hbm, sidx, didx, rows, acc, sem):
        c = lax.axis_index("c")
        s = lax.axis_index("s")
        pltpu.sync_copy(z_hbm, acc.at[pl.ds(s * RPT, RPT)])
        plsc.subcore_barrier()
        base = (c * NS + s) * PERW

        def body(j, carry):
            off = base + j * K
            pltpu.sync_copy(src_hbm.at[pl.ds(off, K)], sidx)
            pltpu.sync_copy(dst_hbm.at[pl.ds(off, K)], didx)
            pltpu.async_copy(h_hbm.at[sidx], rows, sem).wait()
            pltpu.sync_copy(rows, acc.at[didx], add=True)
            return carry

        lax.fori_loop(0, NCHUNK, body, 0)
        plsc.subcore_barrier()
        pltpu.sync_copy(
            acc.at[pl.ds(s * RPT, RPT)], out_hbm.at[c, pl.ds(s * RPT, RPT)]
        )

    return k(h, src, dst, zrows)


def _sc_edge_logits(da, db, src, dst):
    @functools.partial(
        pl.kernel,
        out_type=jax.ShapeDtypeStruct((E,), F32),
        mesh=_mesh(),
        scratch_types=[
            pltpu.VMEM((K,), I32),
            pltpu.VMEM((K,), I32),
            pltpu.VMEM((K,), F32),
            pltpu.VMEM((K,), F32),
            pltpu.VMEM((K,), F32),
            pltpu.SemaphoreType.DMA,
            pltpu.SemaphoreType.DMA,
        ],
    )
    def k(da_hbm, db_hbm, src_hbm, dst_hbm, de_hbm,
          sidx, didx, abuf, bbuf, obuf, sema, semb):
        c = lax.axis_index("c")
        s = lax.axis_index("s")
        base = (c * NS + s) * PERW

        def body(j, carry):
            off = base + j * K
            pltpu.sync_copy(src_hbm.at[pl.ds(off, K)], sidx)
            pltpu.sync_copy(dst_hbm.at[pl.ds(off, K)], didx)
            cpa = pltpu.async_copy(da_hbm.at[sidx], abuf, sema)
            cpb = pltpu.async_copy(db_hbm.at[didx], bbuf, semb)
            cpa.wait()
            cpb.wait()
            for g in range(K // 16):
                sl = pl.ds(g * 16, 16)
                obuf[sl] = abuf[sl] + bbuf[sl]
            pltpu.sync_copy(obuf, de_hbm.at[pl.ds(off, K)])
            return carry

        lax.fori_loop(0, NCHUNK, body, 0)

    return k(da, db, src, dst)


def _sc_deg(wc, wo, src, zdeg):
    @functools.partial(
        pl.kernel,
        out_type=(
            jax.ShapeDtypeStruct((NC * NP,), F32),
            jax.ShapeDtypeStruct((NC * NP,), F32),
        ),
        mesh=_mesh(),
        scratch_types=[
            pltpu.VMEM((K,), I32),
            pltpu.VMEM((K,), F32),
            pltpu.VMEM((K,), F32),
            pltpu.VMEM_SHARED((NP,), F32),
            pltpu.VMEM_SHARED((NP,), F32),
        ],
    )
    def k(wc_hbm, wo_hbm, src_hbm, z_hbm, dc_hbm, do_hbm,
          sidx, wcb, wob, accc, acco):
        c = lax.axis_index("c")
        s = lax.axis_index("s")
        pltpu.sync_copy(z_hbm, accc.at[pl.ds(s * RPT, RPT)])
        pltpu.sync_copy(z_hbm, acco.at[pl.ds(s * RPT, RPT)])
        plsc.subcore_barrier()
        base = (c * NS + s) * PERW

        def body(j, carry):
            off = base + j * K
            pltpu.sync_copy(src_hbm.at[pl.ds(off, K)], sidx)
            pltpu.sync_copy(wc_hbm.at[pl.ds(off, K)], wcb)
            pltpu.sync_copy(wo_hbm.at[pl.ds(off, K)], wob)
            pltpu.sync_copy(wcb, accc.at[sidx], add=True)
            pltpu.sync_copy(wob, acco.at[sidx], add=True)
            return carry

        lax.fori_loop(0, NCHUNK, body, 0)
        plsc.subcore_barrier()
        pltpu.sync_copy(
            accc.at[pl.ds(s * RPT, RPT)],
            dc_hbm.at[pl.ds(c * NP + s * RPT, RPT)],
        )
        pltpu.sync_copy(
            acco.at[pl.ds(s * RPT, RPT)],
            do_hbm.at[pl.ds(c * NP + s * RPT, RPT)],
        )

    return k(wc, wo, src, zdeg)


def _sc_gcn_agg(y, w16, src, dst, zrows):
    @functools.partial(
        pl.kernel,
        out_type=jax.ShapeDtypeStruct((NC, NP, H), F32),
        mesh=_mesh(),
        scratch_types=[
            pltpu.VMEM((K,), I32),
            pltpu.VMEM((K,), I32),
            pltpu.VMEM((K, 16), F32),
            pltpu.VMEM((K, H), F32),
            pltpu.VMEM_SHARED((NP, H), F32),
            pltpu.SemaphoreType.DMA,
        ],
    )
    def k(y_hbm, w_hbm, src_hbm, dst_hbm, z_hbm, out_hbm,
          sidx, didx, wspl, rows, acc, sem):
        c = lax.axis_index("c")
        s = lax.axis_index("s")
        pltpu.sync_copy(z_hbm, acc.at[pl.ds(s * RPT, RPT)])
        plsc.subcore_barrier()
        base = (c * NS + s) * PERW

        def body(j, carry):
            off = base + j * K
            pltpu.sync_copy(src_hbm.at[pl.ds(off, K)], sidx)
            pltpu.sync_copy(dst_hbm.at[pl.ds(off, K)], didx)
            pltpu.sync_copy(w_hbm.at[pl.ds(off, K)], wspl)
            pltpu.async_copy(y_hbm.at[sidx], rows, sem).wait()

            def scale(r, inner):
                wv = wspl[r]
                for q in range(H // 16):
                    sl = pl.ds(q * 16, 16)
                    rows[r, sl] = rows[r, sl] * wv
                return inner

            lax.fori_loop(0, K, scale, 0)
            pltpu.sync_copy(rows, acc.at[didx], add=True)
            return carry

        lax.fori_loop(0, NCHUNK, body, 0)
        plsc.subcore_barrier()
        pltpu.sync_copy(
            acc.at[pl.ds(s * RPT, RPT)], out_hbm.at[c, pl.ds(s * RPT, RPT)]
        )

    return k(y, w16, src, dst, zrows)


def _bn_in(x):
    m = jnp.mean(x, axis=0, keepdims=True)
    xc = x - m
    v = jnp.mean(xc * xc, axis=0, keepdims=True)
    return xc / jnp.sqrt(v + 1e-5) + 1e-4


def _stats(u):
    m = jnp.mean(u, axis=0, keepdims=True)
    s = jnp.sqrt(jnp.var(u, axis=0, keepdims=True) + 1e-5)
    return m, s


def _tc_feat(x, m, s, wf):
    def body(x_ref, m_ref, s_ref, w_ref, o_ref):
        h = (x_ref[...] - m_ref[...]) / s_ref[...] + 1e-4
        o_ref[...] = jnp.maximum(
            _dot3(h, w_ref[...]), 0.0
        )

    return pl.pallas_call(
        body, out_shape=jax.ShapeDtypeStruct((N, H), F32)
    )(x, m, s, wf)


def _tc_gin_a(hs, agg, w1, b1):
    def body(hs_ref, a_ref, w1_ref, b1_ref, u_ref):
        t = hs_ref[...] + a_ref[0, :N, :] + a_ref[1, :N, :]
        u_ref[...] = _dot3(t, w1_ref[...]) + b1_ref[...]

    return pl.pallas_call(
        body, out_shape=jax.ShapeDtypeStruct((N, H), F32)
    )(hs, agg, w1, b1)


def _tc_gin_b(u0, m, s, w2, b2):
    def body(u_ref, m_ref, s_ref, w2_ref, b2_ref, h_ref, h2_ref):
        u = jnp.maximum((u_ref[...] - m_ref[...]) / s_ref[...] + 1e-4, 0.0)
        v = _dot3(u, w2_ref[...]) + b2_ref[...]
        h = jnp.maximum(v, 0.0)
        h_ref[...] = h
        h2_ref[...] = 2.0 * h

    return pl.pallas_call(
        body,
        out_shape=(
            jax.ShapeDtypeStruct((N, H), F32),
            jax.ShapeDtypeStruct((N, H), F32),
        ),
    )(u0, m, s, w2, b2)


def _tc_att(h, wea, bea, wna, bna, wctx, wobj):
    def body(h_ref, wea_ref, bea_ref, wna_ref, bna_ref,
             wc_ref, wo_ref, xcw_ref, xow_ref, da_ref, db_ref):
        h = h_ref[...]
        nl = _dot3(h, wna_ref[...]) + bna_ref[...]
        mx = jnp.max(nl, axis=-1, keepdims=True)
        ex = jnp.exp(nl - mx)
        na = ex / jnp.sum(ex, axis=-1, keepdims=True)
        xc = na[:, 0:1] * h
        xo = na[:, 1:2] * h
        xcw_ref[...] = _dot3(_bn_in(xc), wc_ref[...])
        xow_ref[...] = _dot3(_bn_in(xo), wo_ref[...])
        wea = wea_ref[...]
        eaa = _dot3(h, wea[:H, :]) + bea_ref[...]
        eab = _dot3(h, wea[H:, :])
        da_ref[...] = eaa[:, 1:2] - eaa[:, 0:1]
        db_ref[...] = eab[:, 1:2] - eab[:, 0:1]

    return pl.pallas_call(
        body,
        out_shape=(
            jax.ShapeDtypeStruct((N, H), F32),
            jax.ShapeDtypeStruct((N, H), F32),
            jax.ShapeDtypeStruct((N, 1), F32),
            jax.ShapeDtypeStruct((N, 1), F32),
        ),
    )(h, wea, bea, wna, bna, wctx, wobj)


ER = E // 128


def _tc_edge_sig(de2, row2, col2):
    def body(de_ref, r_ref, c_ref, wc_ref, wo_ref):
        d = de_ref[...]
        keep = (r_ref[...] != c_ref[...]).astype(F32)
        z = 1.0 / (1.0 + jnp.exp(-jnp.abs(d)))
        so = jnp.where(d >= 0.0, z, 1.0 - z)
        wc_ref[...] = (1.0 - so) * keep
        wo_ref[...] = so * keep

    return pl.pallas_call(
        body,
        out_shape=(
            jax.ShapeDtypeStruct((ER, 128), F32),
            jax.ShapeDtypeStruct((ER, 128), F32),
        ),
    )(de2, row2, col2)


def _tc_deg(dcp, dop, xcw, xow):
    def body(dc_in, do_in, xc_ref, xo_ref, yc_ref, yo_ref, dc_ref, do_ref):
        degc = 1.0 + dc_in[0, :N, :] + dc_in[1, :N, :]
        dego = 1.0 + do_in[0, :N, :] + do_in[1, :N, :]
        disc = 1.0 / jnp.sqrt(jnp.maximum(degc, 1e-12))
        diso = 1.0 / jnp.sqrt(jnp.maximum(dego, 1e-12))
        yc_ref[...] = disc * xc_ref[...]
        yo_ref[...] = diso * xo_ref[...]
        dc_ref[...] = disc
        do_ref[...] = diso

    return pl.pallas_call(
        body,
        out_shape=(
            jax.ShapeDtypeStruct((N, H), F32),
            jax.ShapeDtypeStruct((N, H), F32),
            jax.ShapeDtypeStruct((N, 1), F32),
            jax.ShapeDtypeStruct((N, 1), F32),
        ),
    )(dcp, dop, xcw, xow)


def _tc_prepool(accc, yc, dc, bctx, acco, yo, do_, bobj):
    def body(ac_ref, yc_ref, dc_ref, bc_ref, ao_ref, yo_ref, do_ref, bo_ref,
             xc_ref, xo_ref):
        xc_ref[...] = jnp.zeros((NP, H), F32)
        xo_ref[...] = jnp.zeros((NP, H), F32)
        xc_ref[:N, :] = jnp.maximum(
            dc_ref[...] * (ac_ref[0, :N, :] + ac_ref[1, :N, :] + yc_ref[...])
            + bc_ref[...], 0.0
        )
        xo_ref[:N, :] = jnp.maximum(
            do_ref[...] * (ao_ref[0, :N, :] + ao_ref[1, :N, :] + yo_ref[...])
            + bo_ref[...], 0.0
        )

    return pl.pallas_call(
        body,
        out_shape=(
            jax.ShapeDtypeStruct((NP, H), F32),
            jax.ShapeDtypeStruct((NP, H), F32),
        ),
    )(accc, yc, dc, bctx, acco, yo, do_, bobj)


KP = 64
PRW = NP // NW
NPCH = PRW // KP
GPT = G // NS


def _sc_pool(xcp, xop, batchp, zpool):
    @functools.partial(
        pl.kernel,
        out_type=(
            jax.ShapeDtypeStruct((NC, G, H), F32),
            jax.ShapeDtypeStruct((NC, G, H), F32),
        ),
        mesh=_mesh(),
        scratch_types=[
            pltpu.VMEM((KP,), I32),
            pltpu.VMEM((KP, H), F32),
            pltpu.VMEM((KP, H), F32),
            pltpu.VMEM_SHARED((G, H), F32),
            pltpu.VMEM_SHARED((G, H), F32),
        ],
    )
    def k(xc_hbm, xo_hbm, b_hbm, z_hbm, pc_hbm, po_hbm,
          bidx, rowsc, rowso, accc, acco):
        c = lax.axis_index("c")
        s = lax.axis_index("s")
        pltpu.sync_copy(z_hbm, accc.at[pl.ds(s * GPT, GPT)])
        pltpu.sync_copy(z_hbm, acco.at[pl.ds(s * GPT, GPT)])
        plsc.subcore_barrier()
        base = (c * NS + s) * PRW

        def body(j, carry):
            off = base + j * KP
            pltpu.sync_copy(b_hbm.at[pl.ds(off, KP)], bidx)
            pltpu.sync_copy(xc_hbm.at[pl.ds(off, KP)], rowsc)
            pltpu.sync_copy(xo_hbm.at[pl.ds(off, KP)], rowso)
            pltpu.sync_copy(rowsc, accc.at[bidx], add=True)
            pltpu.sync_copy(rowso, acco.at[bidx], add=True)
            return carry

        lax.fori_loop(0, NPCH, body, 0)
        plsc.subcore_barrier()
        pltpu.sync_copy(
            accc.at[pl.ds(s * GPT, GPT)], pc_hbm.at[c, pl.ds(s * GPT, GPT)]
        )
        pltpu.sync_copy(
            acco.at[pl.ds(s * GPT, GPT)], po_hbm.at[c, pl.ds(s * GPT, GPT)]
        )

    return k(xcp, xop, batchp, zpool)


def _tc_readout(z, w1, b1, w2, b2, z_add=None):
    def body(z_ref, za_ref, w1_ref, b1_ref, w2_ref, b2_ref, o_ref):
        z = _bn_in(z_ref[...] + za_ref[...])
        z = jnp.maximum(
            _dot3(z, w1_ref[...]) + b1_ref[...], 0.0
        )
        z = _bn_in(z)
        z = _dot3(z, w2_ref[...]) + b2_ref[...]
        mx = jnp.max(z, axis=-1, keepdims=True)
        ez = jnp.exp(z - mx)
        o_ref[...] = z - mx - jnp.log(jnp.sum(ez, axis=-1, keepdims=True))

    if z_add is None:
        z_add = jnp.zeros((G, H), F32)
    return pl.pallas_call(
        body, out_shape=jax.ShapeDtypeStruct((G, C), F32)
    )(z, z_add, w1, b1, w2, b2)


def _tc_combine(pc2, po2):
    def body(pc_ref, po_ref, pc_o, po_o):
        pc_o[...] = pc_ref[0] + pc_ref[1]
        po_o[...] = po_ref[0] + po_ref[1]

    return pl.pallas_call(
        body,
        out_shape=(
            jax.ShapeDtypeStruct((G, H), F32),
            jax.ShapeDtypeStruct((G, H), F32),
        ),
    )(pc2, po2)


def kernel(x, edge_index, batch, params):
    row = edge_index[0].astype(I32)
    col = edge_index[1].astype(I32)
    p = params

    def b2(v):
        return v.reshape(1, -1)

    zrows = jnp.zeros((RPT, H), F32)
    zdeg = jnp.zeros((RPT,), F32)

    mx_, sx_ = _stats(x)
    h = _tc_feat(x, mx_, sx_, p["W_feat"])
    hs = h
    for i in range(L):
        agg = _sc_segsum(hs, row, col, zrows)
        u0 = _tc_gin_a(hs, agg, p["gin%d_W1" % i], b2(p["gin%d_b1" % i]))
        mu_, su_ = _stats(u0)
        h, hs = _tc_gin_b(
            u0, mu_, su_, p["gin%d_W2" % i], b2(p["gin%d_b2" % i])
        )

    xcw, xow, da2, db2 = _tc_att(
        h, p["W_ea"], b2(p["b_ea"]), p["W_na"], b2(p["b_na"]),
        p["W_ctx"], p["W_obj"],
    )
    de = _sc_edge_logits(da2.reshape(N), db2.reshape(N), row, col)
    wc2, wo2 = _tc_edge_sig(
        de.reshape(ER, 128), row.reshape(ER, 128), col.reshape(ER, 128)
    )
    wc = wc2.reshape(E)
    wo = wo2.reshape(E)
    dcp, dop = _sc_deg(wc, wo, row, zdeg)
    yc, yo, dc, do_ = _tc_deg(
        dcp.reshape(NC, NP, 1), dop.reshape(NC, NP, 1), xcw, xow
    )
    accc = _sc_gcn_agg(yc, jnp.broadcast_to(wc[:, None], (E, 16)), row, col, zrows)
    acco = _sc_gcn_agg(yo, jnp.broadcast_to(wo[:, None], (E, 16)), row, col, zrows)
    xcp, xop = _tc_prepool(
        accc, yc, dc, b2(p["b_ctx"]), acco, yo, do_, b2(p["b_obj"])
    )
    batchp = jnp.concatenate([batch.astype(I32), jnp.zeros((NP - N,), I32)])
    zpool = jnp.zeros((GPT, H), F32)
    pc2, po2 = _sc_pool(xcp, xop, batchp, zpool)
    pc, po = _tc_combine(pc2, po2)

    perm = jax.random.permutation(jax.random.key(1), G)
    oc = _tc_readout(pc, p["fc1_c_W"], b2(p["fc1_c_b"]),
                     p["fc2_c_W"], b2(p["fc2_c_b"]))
    oo = _tc_readout(po, p["fc1_o_W"], b2(p["fc1_o_b"]),
                     p["fc2_o_W"], b2(p["fc2_o_b"]))
    ox = _tc_readout(pc[perm], p["fc1_co_W"], b2(p["fc1_co_b"]),
                     p["fc2_co_W"], b2(p["fc2_co_b"]), z_add=po)
    return (oc, oo, ox)
